# baseline (speedup 1.0000x reference)
import jax
import jax.numpy as jnp
from jax import lax
from jax.experimental import pallas as pl
from jax.experimental.pallas import tpu as pltpu
from jax.experimental.pallas import tpu_sc as plsc

BATCH = 16384
EMB = 64
NROWS = 1000000

_info = plsc.get_sparse_core_info()
_NC, _NS = _info.num_cores, _info.num_subcores
_NW = _NC * _NS
_BPW = BATCH // _NW

_C_OFF = [c * NROWS for c in range(EMB)]


def _build_offsets(idx_ref, ioff_ref):
    def grp(g, c):
        r = idx_ref[pl.ds(16 * g, 16)]
        for cc in range(EMB):
            ioff_ref[pl.ds(cc * _BPW + 16 * g, 16)] = r + _C_OFF[cc]
        return c

    lax.fori_loop(0, _BPW // 16, grp, 0)


def _fire_gather(tbl, ioff_ref, vals, sem):
    flat = tbl.at[0]

    def body(c, s):
        for q in range(_BPW // 128):
            o = c * _BPW + 128 * q
            pltpu.make_async_copy(
                flat.at[ioff_ref.at[pl.ds(o, 128)]],
                vals.at[pl.ds(o, 128)],
                sem,
            ).start()
        return s

    lax.fori_loop(0, EMB, body, 0)


def _wait_gather(vals, sem, out_hbm, base, csem):
    pltpu.make_async_copy(out_hbm.at[pl.ds(0, EMB * _BPW)], vals, sem).wait()

    def body(c, s):
        pltpu.make_async_copy(
            vals.at[pl.ds(c * _BPW, _BPW)],
            out_hbm.at[pl.ds(c * BATCH + base, _BPW)],
            csem,
        ).start()
        return s

    lax.fori_loop(0, EMB, body, 0)
    pltpu.make_async_copy(out_hbm.at[pl.ds(0, EMB * _BPW)], vals, csem).wait()


def _gather_body(user_hbm, item_hbm, ug_t, ig_t, um_t, im_t,
                 eu_out, ei_out, mu_out, mi_out,
                 idx_u, idx_i, ioff, vals0, vals1, sem0, sem1, csem):
    wid = lax.axis_index("s") * _NC + lax.axis_index("c")
    base = wid * _BPW
    pltpu.sync_copy(user_hbm.at[pl.ds(base, _BPW)], idx_u)
    pltpu.sync_copy(item_hbm.at[pl.ds(base, _BPW)], idx_i)

    _build_offsets(idx_u, ioff)
    _fire_gather(ug_t, ioff, vals0, sem0)
    _fire_gather(um_t, ioff, vals1, sem1)
    _wait_gather(vals0, sem0, eu_out, base, csem)
    _wait_gather(vals1, sem1, mu_out, base, csem)

    _build_offsets(idx_i, ioff)
    _fire_gather(ig_t, ioff, vals0, sem0)
    _fire_gather(im_t, ioff, vals1, sem1)
    _wait_gather(vals0, sem0, ei_out, base, csem)
    _wait_gather(vals1, sem1, mi_out, base, csem)


_flat_ty = jax.ShapeDtypeStruct((EMB * BATCH,), jnp.float32)

_sc_gather = pl.kernel(
    _gather_body,
    out_type=(_flat_ty, _flat_ty, _flat_ty, _flat_ty),
    mesh=plsc.VectorSubcoreMesh(core_axis_name="c", subcore_axis_name="s"),
    scratch_types=[
        pltpu.VMEM((_BPW,), jnp.int32),
        pltpu.VMEM((_BPW,), jnp.int32),
        pltpu.VMEM((EMB * _BPW,), jnp.int32),
        pltpu.VMEM((EMB * _BPW,), jnp.float32),
        pltpu.VMEM((EMB * _BPW,), jnp.float32),
        pltpu.SemaphoreType.DMA,
        pltpu.SemaphoreType.DMA,
        pltpu.SemaphoreType.DMA,
    ],
    compiler_params=pltpu.CompilerParams(use_tc_tiling_on_sc=False),
)


_BB = 2048


def _mlp_body(eu, ei, mu, mi, w1a, w1b, b1, w2, b2, w3, b3, wg, wh, bp, out):
    cdims = (((1,), (0,)), ((), ()))
    hp = jax.lax.Precision.HIGHEST
    gmf = eu[...] * ei[...]
    h = lax.dot_general(w1a[...], mu[...], cdims, precision=hp)
    h = h + lax.dot_general(w1b[...], mi[...], cdims, precision=hp)
    h = jnp.maximum(h + b1[...], 0.0)
    h = jnp.maximum(lax.dot_general(w2[...], h, cdims, precision=hp) + b2[...], 0.0)
    h = jnp.maximum(lax.dot_general(w3[...], h, cdims, precision=hp) + b3[...], 0.0)
    pred = jnp.sum(gmf * wg[...], axis=0) + jnp.sum(h * wh[...], axis=0)
    out[...] = pred + bp[0]


def _full(shape):
    nd = len(shape)
    return pl.BlockSpec(shape, lambda i: (0,) * nd)


def kernel(user, item, Ug, Ig, Um, Im, W1, b1, W2, b2, W3, b3, Wp, bp):
    user = user.astype(jnp.int32)
    item = item.astype(jnp.int32)
    eu_f, ei_f, mu_f, mi_f = _sc_gather(user, item, Ug.T, Ig.T, Um.T, Im.T)
    eu_t = eu_f.reshape(EMB, BATCH)
    ei_t = ei_f.reshape(EMB, BATCH)
    mu_t = mu_f.reshape(EMB, BATCH)
    mi_t = mi_f.reshape(EMB, BATCH)

    w1a = W1[:, :EMB]
    w1b = W1[:, EMB:]
    h1 = W1.shape[0]
    h2 = W2.shape[0]
    h3 = W3.shape[0]
    wg = Wp[0, :EMB].reshape(EMB, 1)
    wh = Wp[0, EMB:].reshape(h3, 1)

    grid = BATCH // _BB
    act_spec = pl.BlockSpec((EMB, _BB), lambda i: (0, i))
    out = pl.pallas_call(
        _mlp_body,
        grid=(grid,),
        in_specs=[
            act_spec, act_spec, act_spec, act_spec,
            _full((h1, EMB)), _full((h1, EMB)), _full((h1, 1)),
            _full((h2, h1)), _full((h2, 1)),
            _full((h3, h2)), _full((h3, 1)),
            _full((EMB, 1)), _full((h3, 1)), _full((1,)),
        ],
        out_specs=pl.BlockSpec((_BB,), lambda i: (i,)),
        out_shape=jax.ShapeDtypeStruct((BATCH,), jnp.float32),
    )(eu_t, ei_t, mu_t, mi_t, w1a, w1b, b1.reshape(h1, 1), W2,
      b2.reshape(h2, 1), W3, b3.reshape(h3, 1), wg, wh, bp)
    return out


# speedup vs baseline: 10.6655x; 10.6655x over previous
import jax
import jax.numpy as jnp
from jax import lax
from jax.experimental import pallas as pl
from jax.experimental.pallas import tpu as pltpu
from jax.experimental.pallas import tpu_sc as plsc

BATCH = 16384
EMB = 64
NROWS = 1000000
_RCH = 2048
_NCH = (NROWS + _RCH - 1) // _RCH

_info = plsc.get_sparse_core_info()
_NC, _NS = _info.num_cores, _info.num_subcores
_NW = _NC * _NS
_BPW = BATCH // _NW


def _repack_body(src, dst):
    y = src[...].T
    dst[...] = jnp.concatenate([y[: _RCH // 2], y[_RCH // 2:]], axis=1)


_repack = pl.pallas_call(
    _repack_body,
    grid=(_NCH,),
    in_specs=[pl.BlockSpec((EMB, _RCH), lambda j: (0, j))],
    out_specs=pl.BlockSpec((_RCH // 2, 2 * EMB), lambda j: (j, 0)),
    out_shape=jax.ShapeDtypeStruct((_NCH * _RCH // 2, 2 * EMB), jnp.float32),
)


def _gather_body(idx_hbm, tbl, out_hbm, idx_v, idxh, vals, sem):
    wid = lax.axis_index("s") * _NC + lax.axis_index("c")
    base = wid * _BPW
    pltpu.sync_copy(idx_hbm.at[pl.ds(base, _BPW)], idx_v)

    def grp(g, c):
        r = idx_v[pl.ds(16 * g, 16)]
        idxh[pl.ds(16 * g, 16)] = ((r >> 11) << 10) + (r & 1023)
        return c

    lax.fori_loop(0, _BPW // 16, grp, 0)
    pltpu.async_copy(tbl.at[idxh], vals, sem).wait()
    pltpu.sync_copy(vals, out_hbm.at[pl.ds(base, _BPW)])


_sc_gather = pl.kernel(
    _gather_body,
    out_type=jax.ShapeDtypeStruct((BATCH, 2 * EMB), jnp.float32),
    mesh=plsc.VectorSubcoreMesh(core_axis_name="c", subcore_axis_name="s"),
    scratch_types=[
        pltpu.VMEM((_BPW,), jnp.int32),
        pltpu.VMEM((_BPW,), jnp.int32),
        pltpu.VMEM((_BPW, 2 * EMB), jnp.float32),
        pltpu.SemaphoreType.DMA,
    ],
    compiler_params=pltpu.CompilerParams(use_tc_tiling_on_sc=False),
)


_BB = 2048


def _mlp_body(u2, i2, eu2, ei2, mu2, mi2,
              w1a, w1b, b1, w2, b2, w3, b3, wg, wh, bp, out):
    pu = ((u2[...] >> 10) & 1) == 1
    pi = ((i2[...] >> 10) & 1) == 1
    eu = jnp.where(pu, eu2[:, EMB:], eu2[:, :EMB])
    mu = jnp.where(pu, mu2[:, EMB:], mu2[:, :EMB])
    ei = jnp.where(pi, ei2[:, EMB:], ei2[:, :EMB])
    mi = jnp.where(pi, mi2[:, EMB:], mi2[:, :EMB])

    cdims = (((1,), (1,)), ((), ()))
    hp = jax.lax.Precision.HIGHEST
    gmf = eu * ei
    h = lax.dot_general(mu, w1a[...], cdims, precision=hp)
    h = h + lax.dot_general(mi, w1b[...], cdims, precision=hp)
    h = jnp.maximum(h + b1[...], 0.0)
    h = jnp.maximum(lax.dot_general(h, w2[...], cdims, precision=hp) + b2[...], 0.0)
    h = jnp.maximum(lax.dot_general(h, w3[...], cdims, precision=hp) + b3[...], 0.0)
    pred = jnp.sum(gmf * wg[...], axis=1) + jnp.sum(h * wh[...], axis=1)
    out[...] = pred + bp[0]


def _full(shape):
    nd = len(shape)
    return pl.BlockSpec(shape, lambda i: (0,) * nd)


def kernel(user, item, Ug, Ig, Um, Im, W1, b1, W2, b2, W3, b3, Wp, bp):
    user = user.astype(jnp.int32)
    item = item.astype(jnp.int32)

    eu2 = _sc_gather(user, _repack(Ug.T))
    ei2 = _sc_gather(item, _repack(Ig.T))
    mu2 = _sc_gather(user, _repack(Um.T))
    mi2 = _sc_gather(item, _repack(Im.T))

    w1a = W1[:, :EMB]
    w1b = W1[:, EMB:]
    h1 = W1.shape[0]
    h2 = W2.shape[0]
    h3 = W3.shape[0]
    wg = Wp[:, :EMB]
    wh = Wp[:, EMB:]

    grid = BATCH // _BB
    pair_spec = pl.BlockSpec((_BB, 2 * EMB), lambda i: (i, 0))
    idx_spec = pl.BlockSpec((_BB, 1), lambda i: (i, 0))
    out = pl.pallas_call(
        _mlp_body,
        grid=(grid,),
        in_specs=[
            idx_spec, idx_spec,
            pair_spec, pair_spec, pair_spec, pair_spec,
            _full((h1, EMB)), _full((h1, EMB)), _full((1, h1)),
            _full((h2, h1)), _full((1, h2)),
            _full((h3, h2)), _full((1, h3)),
            _full((1, EMB)), _full((1, h3)), _full((1,)),
        ],
        out_specs=pl.BlockSpec((_BB,), lambda i: (i,)),
        out_shape=jax.ShapeDtypeStruct((BATCH,), jnp.float32),
    )(user.reshape(BATCH, 1), item.reshape(BATCH, 1),
      eu2, ei2, mu2, mi2, w1a, w1b, b1.reshape(1, h1), W2,
      b2.reshape(1, h2), W3, b3.reshape(1, h3), wg, wh, bp)
    return out


# speedup vs baseline: 17.7870x; 1.6677x over previous
import jax
import jax.numpy as jnp
from jax import lax
from jax.experimental import pallas as pl
from jax.experimental.pallas import tpu as pltpu
from jax.experimental.pallas import tpu_sc as plsc

BATCH = 16384
EMB = 64
NROWS = 1000000
_RCH = 8192
_NCH = (NROWS + _RCH - 1) // _RCH
_LOG_RCH = _RCH.bit_length() - 1
_HMASK = _RCH // 2 - 1

_info = plsc.get_sparse_core_info()
_NC, _NS = _info.num_cores, _info.num_subcores
_NW = _NC * _NS
_BPW = BATCH // _NW


def _repack_body(src, dst):
    y = src[...].T
    dst[...] = jnp.concatenate([y[: _RCH // 2], y[_RCH // 2:]], axis=1)


_repack = pl.pallas_call(
    _repack_body,
    grid=(_NCH,),
    in_specs=[pl.BlockSpec((EMB, _RCH), lambda j: (0, j))],
    out_specs=pl.BlockSpec((_RCH // 2, 2 * EMB), lambda j: (j, 0)),
    out_shape=jax.ShapeDtypeStruct((_NCH * _RCH // 2, 2 * EMB), jnp.float32),
)


def _gather_body(idx_hbm, tbl, out_hbm, idx_v, idxh, vals, sem):
    wid = lax.axis_index("s") * _NC + lax.axis_index("c")
    base = wid * _BPW
    pltpu.sync_copy(idx_hbm.at[pl.ds(base, _BPW)], idx_v)

    def grp(g, c):
        r = idx_v[pl.ds(16 * g, 16)]
        idxh[pl.ds(16 * g, 16)] = (
            (r >> _LOG_RCH) << (_LOG_RCH - 1)) + (r & _HMASK)
        return c

    lax.fori_loop(0, _BPW // 16, grp, 0)
    pltpu.async_copy(tbl.at[idxh], vals, sem).wait()
    pltpu.sync_copy(vals, out_hbm.at[pl.ds(base, _BPW)])


_sc_gather = pl.kernel(
    _gather_body,
    out_type=jax.ShapeDtypeStruct((BATCH, 2 * EMB), jnp.float32),
    mesh=plsc.VectorSubcoreMesh(core_axis_name="c", subcore_axis_name="s"),
    scratch_types=[
        pltpu.VMEM((_BPW,), jnp.int32),
        pltpu.VMEM((_BPW,), jnp.int32),
        pltpu.VMEM((_BPW, 2 * EMB), jnp.float32),
        pltpu.SemaphoreType.DMA,
    ],
    compiler_params=pltpu.CompilerParams(use_tc_tiling_on_sc=False),
)


_BB = 2048


def _mlp_body(u2, i2, eu2, ei2, mu2, mi2,
              w1a, w1b, b1, w2, b2, w3, b3, wg, wh, bp, out):
    pu = ((u2[...] >> (_LOG_RCH - 1)) & 1) == 1
    pi = ((i2[...] >> (_LOG_RCH - 1)) & 1) == 1
    eu = jnp.where(pu, eu2[:, EMB:], eu2[:, :EMB])
    mu = jnp.where(pu, mu2[:, EMB:], mu2[:, :EMB])
    ei = jnp.where(pi, ei2[:, EMB:], ei2[:, :EMB])
    mi = jnp.where(pi, mi2[:, EMB:], mi2[:, :EMB])

    cdims = (((1,), (1,)), ((), ()))
    hp = jax.lax.Precision.HIGHEST
    gmf = eu * ei
    h = lax.dot_general(mu, w1a[...], cdims, precision=hp)
    h = h + lax.dot_general(mi, w1b[...], cdims, precision=hp)
    h = jnp.maximum(h + b1[...], 0.0)
    h = jnp.maximum(lax.dot_general(h, w2[...], cdims, precision=hp) + b2[...], 0.0)
    h = jnp.maximum(lax.dot_general(h, w3[...], cdims, precision=hp) + b3[...], 0.0)
    pred = jnp.sum(gmf * wg[...], axis=1) + jnp.sum(h * wh[...], axis=1)
    out[...] = pred + bp[0]


def _full(shape):
    nd = len(shape)
    return pl.BlockSpec(shape, lambda i: (0,) * nd)


def kernel(user, item, Ug, Ig, Um, Im, W1, b1, W2, b2, W3, b3, Wp, bp):
    user = user.astype(jnp.int32)
    item = item.astype(jnp.int32)

    eu2 = _sc_gather(user, _repack(Ug.T))
    ei2 = _sc_gather(item, _repack(Ig.T))
    mu2 = _sc_gather(user, _repack(Um.T))
    mi2 = _sc_gather(item, _repack(Im.T))

    w1a = W1[:, :EMB]
    w1b = W1[:, EMB:]
    h1 = W1.shape[0]
    h2 = W2.shape[0]
    h3 = W3.shape[0]
    wg = Wp[:, :EMB]
    wh = Wp[:, EMB:]

    grid = BATCH // _BB
    pair_spec = pl.BlockSpec((_BB, 2 * EMB), lambda i: (i, 0))
    idx_spec = pl.BlockSpec((_BB, 1), lambda i: (i, 0))
    out = pl.pallas_call(
        _mlp_body,
        grid=(grid,),
        in_specs=[
            idx_spec, idx_spec,
            pair_spec, pair_spec, pair_spec, pair_spec,
            _full((h1, EMB)), _full((h1, EMB)), _full((1, h1)),
            _full((h2, h1)), _full((1, h2)),
            _full((h3, h2)), _full((1, h3)),
            _full((1, EMB)), _full((1, h3)), _full((1,)),
        ],
        out_specs=pl.BlockSpec((_BB,), lambda i: (i,)),
        out_shape=jax.ShapeDtypeStruct((BATCH,), jnp.float32),
    )(user.reshape(BATCH, 1), item.reshape(BATCH, 1),
      eu2, ei2, mu2, mi2, w1a, w1b, b1.reshape(1, h1), W2,
      b2.reshape(1, h2), W3, b3.reshape(1, h3), wg, wh, bp)
    return out


# speedup vs baseline: 21.4294x; 1.2048x over previous
import jax
import jax.numpy as jnp
from jax import lax
from jax.experimental import pallas as pl
from jax.experimental.pallas import tpu as pltpu
from jax.experimental.pallas import tpu_sc as plsc

BATCH = 16384
EMB = 64
NROWS = 1000000
_RCH = 32768
_NCH = (NROWS + _RCH - 1) // _RCH
_LOG_RCH = _RCH.bit_length() - 1
_HMASK = _RCH // 2 - 1

_info = plsc.get_sparse_core_info()
_NC, _NS = _info.num_cores, _info.num_subcores
_NW = _NC * _NS
_BPW = BATCH // _NW


def _repack_body(src, dst):
    y = src[...].T
    dst[...] = jnp.concatenate([y[: _RCH // 2], y[_RCH // 2:]], axis=1)


_repack = pl.pallas_call(
    _repack_body,
    grid=(_NCH,),
    in_specs=[pl.BlockSpec((EMB, _RCH), lambda j: (0, j))],
    out_specs=pl.BlockSpec((_RCH // 2, 2 * EMB), lambda j: (j, 0)),
    out_shape=jax.ShapeDtypeStruct((_NCH * _RCH // 2, 2 * EMB), jnp.float32),
)


def _gather_body(idx_hbm, tbl, out_hbm, idx_v, idxh, vals, sem):
    wid = lax.axis_index("s") * _NC + lax.axis_index("c")
    base = wid * _BPW
    pltpu.sync_copy(idx_hbm.at[pl.ds(base, _BPW)], idx_v)

    def grp(g, c):
        r = idx_v[pl.ds(16 * g, 16)]
        idxh[pl.ds(16 * g, 16)] = (
            (r >> _LOG_RCH) << (_LOG_RCH - 1)) + (r & _HMASK)
        return c

    lax.fori_loop(0, _BPW // 16, grp, 0)
    pltpu.async_copy(tbl.at[idxh], vals, sem).wait()
    pltpu.sync_copy(vals, out_hbm.at[pl.ds(base, _BPW)])


_sc_gather = pl.kernel(
    _gather_body,
    out_type=jax.ShapeDtypeStruct((BATCH, 2 * EMB), jnp.float32),
    mesh=plsc.VectorSubcoreMesh(core_axis_name="c", subcore_axis_name="s"),
    scratch_types=[
        pltpu.VMEM((_BPW,), jnp.int32),
        pltpu.VMEM((_BPW,), jnp.int32),
        pltpu.VMEM((_BPW, 2 * EMB), jnp.float32),
        pltpu.SemaphoreType.DMA,
    ],
    compiler_params=pltpu.CompilerParams(use_tc_tiling_on_sc=False),
)


_BB = 2048


def _mlp_body(u2, i2, eu2, ei2, mu2, mi2,
              w1a, w1b, b1, w2, b2, w3, b3, wg, wh, bp, out):
    pu = ((u2[...] >> (_LOG_RCH - 1)) & 1) == 1
    pi = ((i2[...] >> (_LOG_RCH - 1)) & 1) == 1
    eu = jnp.where(pu, eu2[:, EMB:], eu2[:, :EMB])
    mu = jnp.where(pu, mu2[:, EMB:], mu2[:, :EMB])
    ei = jnp.where(pi, ei2[:, EMB:], ei2[:, :EMB])
    mi = jnp.where(pi, mi2[:, EMB:], mi2[:, :EMB])

    cdims = (((1,), (1,)), ((), ()))
    hp = jax.lax.Precision.HIGHEST
    gmf = eu * ei
    h = lax.dot_general(mu, w1a[...], cdims, precision=hp)
    h = h + lax.dot_general(mi, w1b[...], cdims, precision=hp)
    h = jnp.maximum(h + b1[...], 0.0)
    h = jnp.maximum(lax.dot_general(h, w2[...], cdims, precision=hp) + b2[...], 0.0)
    h = jnp.maximum(lax.dot_general(h, w3[...], cdims, precision=hp) + b3[...], 0.0)
    pred = jnp.sum(gmf * wg[...], axis=1) + jnp.sum(h * wh[...], axis=1)
    out[...] = pred + bp[0]


def _full(shape):
    nd = len(shape)
    return pl.BlockSpec(shape, lambda i: (0,) * nd)


def kernel(user, item, Ug, Ig, Um, Im, W1, b1, W2, b2, W3, b3, Wp, bp):
    user = user.astype(jnp.int32)
    item = item.astype(jnp.int32)

    eu2 = _sc_gather(user, _repack(Ug.T))
    ei2 = _sc_gather(item, _repack(Ig.T))
    mu2 = _sc_gather(user, _repack(Um.T))
    mi2 = _sc_gather(item, _repack(Im.T))

    w1a = W1[:, :EMB]
    w1b = W1[:, EMB:]
    h1 = W1.shape[0]
    h2 = W2.shape[0]
    h3 = W3.shape[0]
    wg = Wp[:, :EMB]
    wh = Wp[:, EMB:]

    grid = BATCH // _BB
    pair_spec = pl.BlockSpec((_BB, 2 * EMB), lambda i: (i, 0))
    idx_spec = pl.BlockSpec((_BB, 1), lambda i: (i, 0))
    out = pl.pallas_call(
        _mlp_body,
        grid=(grid,),
        in_specs=[
            idx_spec, idx_spec,
            pair_spec, pair_spec, pair_spec, pair_spec,
            _full((h1, EMB)), _full((h1, EMB)), _full((1, h1)),
            _full((h2, h1)), _full((1, h2)),
            _full((h3, h2)), _full((1, h3)),
            _full((1, EMB)), _full((1, h3)), _full((1,)),
        ],
        out_specs=pl.BlockSpec((_BB,), lambda i: (i,)),
        out_shape=jax.ShapeDtypeStruct((BATCH,), jnp.float32),
    )(user.reshape(BATCH, 1), item.reshape(BATCH, 1),
      eu2, ei2, mu2, mi2, w1a, w1b, b1.reshape(1, h1), W2,
      b2.reshape(1, h2), W3, b3.reshape(1, h3), wg, wh, bp)
    return out
